# baseline (speedup 1.0000x reference)
import functools

import jax
import jax.numpy as jnp
from jax import lax
from jax.experimental import pallas as pl
from jax.experimental.pallas import tpu as pltpu
from jax.experimental.pallas import tpu_sc as plsc

V, D, C, B, N = 1000000, 64, 16, 16384, 20

NC, NS = 2, 16
NW = NC * NS
CH = 128
BPW = B // NW
POS_CH = BPW // CH
NEG_CH = BPW * N // CH
W2 = 2 * D


def _sc_body(pu_hbm, pv_hbm, ng_hbm, u_hbm, v_hbm,
             eu_out, ps_out, ns_out,
             u_wide, ring, pu_v, pv_v, ng_v, pos_sb, neg_sb, gsem, wsem):
    wid = lax.axis_index("s") * NC + lax.axis_index("c")
    pltpu.sync_copy(pu_hbm.at[wid], pu_v)
    pltpu.sync_copy(pv_hbm.at[wid], pv_v)
    pltpu.sync_copy(ng_hbm.at[wid], ng_v)

    lane = lax.iota(jnp.int32, 16)
    last = lane == 15

    hu = [pltpu.async_copy(u_hbm.at[pu_v.at[j]],
                           u_wide.at[pl.ds(j * CH, CH)], gsem)
          for j in range(POS_CH)]
    for c in hu:
        c.wait()
    wu = pltpu.async_copy(u_wide, eu_out.at[pl.ds(wid * BPW, BPW)], wsem)

    def dot_store(sb, store_idx, b_row, buf, parity, t):
        acc = u_wide[b_row, pl.ds(0, 16)] * buf[parity, t, pl.ds(0, 16)]
        for k in range(1, 4):
            acc += (u_wide[b_row, pl.ds(k * 16, 16)]
                    * buf[parity, t, pl.ds(k * 16, 16)])
        tot = plsc.cumsum(acc)
        plsc.store_scatter(sb, [jnp.full((16,), store_idx, jnp.int32)], tot,
                           mask=last)

    pv0 = pltpu.async_copy(v_hbm.at[pv_v.at[0]], ring.at[0], gsem)

    def pos_chunk(j, carry):
        par = lax.rem(j, 2)
        pltpu.make_async_copy(v_hbm.at[pl.ds(0, CH)], ring.at[par],
                              gsem).wait()

        @pl.when(j < POS_CH - 1)
        def _():
            pltpu.async_copy(v_hbm.at[pv_v.at[j + 1]],
                             ring.at[lax.rem(j + 1, 2)], gsem)

        def group(g, carry2):
            t0 = g * 16
            for s in range(16):
                t = t0 + s
                b = j * CH + t
                dot_store(pos_sb, b, b, ring, par, t)
            return carry2

        lax.fori_loop(0, CH // 16, group, 0)
        return carry

    lax.fori_loop(0, POS_CH, pos_chunk, 0)
    del pv0

    ng0 = pltpu.async_copy(v_hbm.at[ng_v.at[0]], ring.at[0], gsem)

    def neg_chunk(j, carry):
        par = lax.rem(j, 2)
        pltpu.make_async_copy(v_hbm.at[pl.ds(0, CH)], ring.at[par],
                              gsem).wait()

        @pl.when(j < NEG_CH - 1)
        def _():
            pltpu.async_copy(v_hbm.at[ng_v.at[j + 1]],
                             ring.at[lax.rem(j + 1, 2)], gsem)

        def group(g, carry2):
            r0 = j * CH + g * 16
            for s in range(16):
                r = r0 + s
                b = lax.div(r, N)
                dot_store(neg_sb, r, b, ring, par, r - j * CH)
            return carry2

        lax.fori_loop(0, CH // 16, group, 0)
        return carry

    lax.fori_loop(0, NEG_CH, neg_chunk, 0)
    del ng0

    pltpu.sync_copy(pos_sb, ps_out.at[wid])
    pltpu.sync_copy(neg_sb, ns_out.at[wid])
    wu.wait()


_sc_fused = functools.partial(
    pl.kernel,
    compiler_params=pltpu.CompilerParams(needs_layout_passes=False),
    out_type=[
        jax.ShapeDtypeStruct((B, W2), jnp.float32),
        jax.ShapeDtypeStruct((NW, BPW), jnp.float32),
        jax.ShapeDtypeStruct((NW, BPW * N), jnp.float32),
    ],
    mesh=plsc.VectorSubcoreMesh(core_axis_name="c", subcore_axis_name="s"),
    scratch_types=[
        pltpu.VMEM((BPW, W2), jnp.float32),
        pltpu.VMEM((2, CH, W2), jnp.float32),
        pltpu.VMEM((POS_CH, CH), jnp.int32),
        pltpu.VMEM((POS_CH, CH), jnp.int32),
        pltpu.VMEM((NEG_CH, CH), jnp.int32),
        pltpu.VMEM((BPW,), jnp.float32),
        pltpu.VMEM((BPW * N,), jnp.float32),
        pltpu.SemaphoreType.DMA,
        pltpu.SemaphoreType.DMA,
    ],
)(_sc_body)


def _tc_body(eu_ref, ps_ref, ns_ref, w_ref, b_ref, acc_ref, dur_ref):
    s = jnp.clip(ps_ref[...], -10.0, 10.0)
    pos = jnp.log1p(jnp.exp(-s))
    ns = jnp.clip(ns_ref[...], -10.0, 10.0)
    neg = jnp.log1p(jnp.exp(ns))
    total = (jnp.sum(pos) + jnp.sum(neg)) * (1.0 / B)
    acc_ref[...] = total[None, None]
    dur_ref[...] = lax.dot_general(
        eu_ref[:, pl.ds(0, D)], w_ref[...], (((1,), (1,)), ((), ())),
        preferred_element_type=jnp.float32) + b_ref[...]


def _tc_math(eu_wide, pos_s, neg_s, W, b2):
    return pl.pallas_call(
        _tc_body,
        grid=(1,),
        in_specs=[
            pl.BlockSpec((B, W2), lambda i: (0, 0)),
            pl.BlockSpec((NW, BPW), lambda i: (0, 0)),
            pl.BlockSpec((NW, BPW * N), lambda i: (0, 0)),
            pl.BlockSpec((C, D), lambda i: (0, 0)),
            pl.BlockSpec((1, C), lambda i: (0, 0)),
        ],
        out_specs=[
            pl.BlockSpec((1, 1), lambda i: (0, 0)),
            pl.BlockSpec((B, C), lambda i: (0, 0)),
        ],
        out_shape=[
            jax.ShapeDtypeStruct((1, 1), jnp.float32),
            jax.ShapeDtypeStruct((B, C), jnp.float32),
        ],
    )(eu_wide, pos_s, neg_s, W, b2)


def kernel(pos_u, pos_v, neg_v, predict_fix, u_emb, v_emb, W, b):
    u128 = jnp.pad(u_emb, ((0, 0), (0, D)))
    v128 = jnp.pad(v_emb, ((0, 0), (0, D)))
    pu = pos_u.astype(jnp.int32).reshape(NW, POS_CH, CH)
    pv = pos_v.astype(jnp.int32).reshape(NW, POS_CH, CH)
    ng = neg_v.astype(jnp.int32).reshape(NW, NEG_CH, CH)

    eu_wide, pos_s, neg_s = _sc_fused(pu, pv, ng, u128, v128)

    acc, duration = _tc_math(eu_wide, pos_s, neg_s, W, b.reshape(1, C))
    return acc[0, 0], duration


# speedup vs baseline: 1.1012x; 1.1012x over previous
import functools

import jax
import jax.numpy as jnp
from jax import lax
from jax.experimental import pallas as pl
from jax.experimental.pallas import tpu as pltpu
from jax.experimental.pallas import tpu_sc as plsc

V, D, C, B, N = 1000000, 64, 16, 16384, 20

NC, NS = 2, 16
NW = NC * NS
CH = 128
BPW = B // NW
POS_CH = BPW // CH
NEG_CH = BPW * N // CH
W2 = 2 * D


def _sc_body(pu_hbm, pv_hbm, ng_hbm, u_hbm, v_hbm,
             eu_out, ps_out, ns_out,
             u_wide, ring, pu_v, pv_v, ng_v, pos_sb, neg_sb, gsem, wsem):
    wid = lax.axis_index("s") * NC + lax.axis_index("c")
    pltpu.sync_copy(pu_hbm.at[wid], pu_v)
    pltpu.sync_copy(pv_hbm.at[wid], pv_v)
    pltpu.sync_copy(ng_hbm.at[wid], ng_v)

    lane = lax.iota(jnp.int32, 16)
    last = lane == 15

    hu = [pltpu.async_copy(u_hbm.at[pu_v.at[j]],
                           u_wide.at[pl.ds(j * CH, CH)], gsem)
          for j in range(POS_CH)]
    for c in hu:
        c.wait()
    wu = pltpu.async_copy(u_wide, eu_out.at[pl.ds(wid * BPW, BPW)], wsem)

    def dot_store(sb, store_idx, b_row, buf, parity, t):
        acc = u_wide[b_row, pl.ds(0, 16)] * buf[parity, t, pl.ds(D, 16)]
        for k in range(1, 4):
            acc += (u_wide[b_row, pl.ds(k * 16, 16)]
                    * buf[parity, t, pl.ds(D + k * 16, 16)])
        tot = plsc.cumsum(acc)
        plsc.store_scatter(sb, [jnp.full((16,), store_idx, jnp.int32)], tot,
                           mask=last)

    pv0 = pltpu.async_copy(v_hbm.at[pv_v.at[0]], ring.at[0], gsem)

    def pos_chunk(j, carry):
        par = lax.rem(j, 2)
        pltpu.make_async_copy(v_hbm.at[pl.ds(0, CH)], ring.at[par],
                              gsem).wait()

        @pl.when(j < POS_CH - 1)
        def _():
            pltpu.async_copy(v_hbm.at[pv_v.at[j + 1]],
                             ring.at[lax.rem(j + 1, 2)], gsem)

        def group(g, carry2):
            t0 = g * 16
            for s in range(16):
                t = t0 + s
                b = j * CH + t
                dot_store(pos_sb, b, b, ring, par, t)
            return carry2

        lax.fori_loop(0, CH // 16, group, 0)
        return carry

    lax.fori_loop(0, POS_CH, pos_chunk, 0)
    del pv0

    ng0 = pltpu.async_copy(v_hbm.at[ng_v.at[0]], ring.at[0], gsem)

    def neg_chunk(j, carry):
        par = lax.rem(j, 2)
        pltpu.make_async_copy(v_hbm.at[pl.ds(0, CH)], ring.at[par],
                              gsem).wait()

        @pl.when(j < NEG_CH - 1)
        def _():
            pltpu.async_copy(v_hbm.at[ng_v.at[j + 1]],
                             ring.at[lax.rem(j + 1, 2)], gsem)

        def group(g, carry2):
            r0 = j * CH + g * 16
            for s in range(16):
                r = r0 + s
                b = lax.div(r, N)
                dot_store(neg_sb, r, b, ring, par, r - j * CH)
            return carry2

        lax.fori_loop(0, CH // 16, group, 0)
        return carry

    lax.fori_loop(0, NEG_CH, neg_chunk, 0)
    del ng0

    pltpu.sync_copy(pos_sb, ps_out.at[wid])
    pltpu.sync_copy(neg_sb, ns_out.at[wid])
    wu.wait()


_sc_fused = functools.partial(
    pl.kernel,
    compiler_params=pltpu.CompilerParams(needs_layout_passes=False),
    out_type=[
        jax.ShapeDtypeStruct((B, W2), jnp.float32),
        jax.ShapeDtypeStruct((NW, BPW), jnp.float32),
        jax.ShapeDtypeStruct((NW, BPW * N), jnp.float32),
    ],
    mesh=plsc.VectorSubcoreMesh(core_axis_name="c", subcore_axis_name="s"),
    scratch_types=[
        pltpu.VMEM((BPW, W2), jnp.float32),
        pltpu.VMEM((2, CH, W2), jnp.float32),
        pltpu.VMEM((POS_CH, CH), jnp.int32),
        pltpu.VMEM((POS_CH, CH), jnp.int32),
        pltpu.VMEM((NEG_CH, CH), jnp.int32),
        pltpu.VMEM((BPW,), jnp.float32),
        pltpu.VMEM((BPW * N,), jnp.float32),
        pltpu.SemaphoreType.DMA,
        pltpu.SemaphoreType.DMA,
    ],
)(_sc_body)


_TR_BLK = 1024


def _tr_body(ut_ref, vt_ref, uv_ref):
    uT = jnp.transpose(ut_ref[...], (1, 0))
    vT = jnp.transpose(vt_ref[...], (1, 0))
    uv_ref[...] = jnp.concatenate([uT, vT], axis=1)


def _build_uv(u_emb, v_emb):
    ut = u_emb.T
    vt = v_emb.T
    grid = (pl.cdiv(V, _TR_BLK),)
    return pl.pallas_call(
        _tr_body,
        grid=grid,
        in_specs=[
            pl.BlockSpec((D, _TR_BLK), lambda i: (0, i)),
            pl.BlockSpec((D, _TR_BLK), lambda i: (0, i)),
        ],
        out_specs=pl.BlockSpec((_TR_BLK, W2), lambda i: (i, 0)),
        out_shape=jax.ShapeDtypeStruct((V, W2), jnp.float32),
    )(ut, vt)


def _tc_body(eu_ref, ps_ref, ns_ref, w_ref, b_ref, acc_ref, dur_ref):
    s = jnp.clip(ps_ref[...], -10.0, 10.0)
    pos = jnp.log1p(jnp.exp(-s))
    ns = jnp.clip(ns_ref[...], -10.0, 10.0)
    neg = jnp.log1p(jnp.exp(ns))
    total = (jnp.sum(pos) + jnp.sum(neg)) * (1.0 / B)
    acc_ref[...] = total[None, None]
    dur_ref[...] = lax.dot_general(
        eu_ref[:, pl.ds(0, D)], w_ref[...], (((1,), (1,)), ((), ())),
        preferred_element_type=jnp.float32) + b_ref[...]


def _tc_math(eu_wide, pos_s, neg_s, W, b2):
    return pl.pallas_call(
        _tc_body,
        grid=(1,),
        in_specs=[
            pl.BlockSpec((B, W2), lambda i: (0, 0)),
            pl.BlockSpec((NW, BPW), lambda i: (0, 0)),
            pl.BlockSpec((NW, BPW * N), lambda i: (0, 0)),
            pl.BlockSpec((C, D), lambda i: (0, 0)),
            pl.BlockSpec((1, C), lambda i: (0, 0)),
        ],
        out_specs=[
            pl.BlockSpec((1, 1), lambda i: (0, 0)),
            pl.BlockSpec((B, C), lambda i: (0, 0)),
        ],
        out_shape=[
            jax.ShapeDtypeStruct((1, 1), jnp.float32),
            jax.ShapeDtypeStruct((B, C), jnp.float32),
        ],
    )(eu_wide, pos_s, neg_s, W, b2)


def kernel(pos_u, pos_v, neg_v, predict_fix, u_emb, v_emb, W, b):
    uv = _build_uv(u_emb, v_emb)
    pu = pos_u.astype(jnp.int32).reshape(NW, POS_CH, CH)
    pv = pos_v.astype(jnp.int32).reshape(NW, POS_CH, CH)
    ng = neg_v.astype(jnp.int32).reshape(NW, NEG_CH, CH)

    eu_wide, pos_s, neg_s = _sc_fused(pu, pv, ng, uv, uv)

    acc, duration = _tc_math(eu_wide, pos_s, neg_s, W, b.reshape(1, C))
    return acc[0, 0], duration


# speedup vs baseline: 1.1107x; 1.0087x over previous
import functools

import jax
import jax.numpy as jnp
from jax import lax
from jax.experimental import pallas as pl
from jax.experimental.pallas import tpu as pltpu
from jax.experimental.pallas import tpu_sc as plsc

V, D, C, B, N = 1000000, 64, 16, 16384, 20

NC, NS = 2, 16
NW = NC * NS
CH = 128
BPW = B // NW
POS_CH = BPW // CH
NEG_CH = BPW * N // CH
W2 = 2 * D


def _sc_body(pu_hbm, pv_hbm, ng_hbm, u_hbm, v_hbm,
             eu_out, ps_out, ns_out,
             u_wide, ring, pu_v, pv_v, ng_v, pos_sb, neg_sb, gsem, wsem):
    wid = lax.axis_index("s") * NC + lax.axis_index("c")
    pltpu.sync_copy(pu_hbm.at[wid], pu_v)
    pltpu.sync_copy(pv_hbm.at[wid], pv_v)
    pltpu.sync_copy(ng_hbm.at[wid], ng_v)

    lane = lax.iota(jnp.int32, 16)
    last = lane == 15

    hu = [pltpu.async_copy(u_hbm.at[pu_v.at[j]],
                           u_wide.at[pl.ds(j * CH, CH)], gsem)
          for j in range(POS_CH)]
    for c in hu:
        c.wait()
    wu = pltpu.async_copy(u_wide, eu_out.at[pl.ds(wid * BPW, BPW)], wsem)

    def dot_store(sb, store_idx, b_row, buf, parity, t):
        acc = u_wide[b_row, pl.ds(0, 16)] * buf[parity, t, pl.ds(D, 16)]
        for k in range(1, 4):
            acc += (u_wide[b_row, pl.ds(k * 16, 16)]
                    * buf[parity, t, pl.ds(D + k * 16, 16)])
        tot = plsc.cumsum(acc)
        plsc.store_scatter(sb, [jnp.full((16,), store_idx, jnp.int32)], tot,
                           mask=last)

    pv0 = pltpu.async_copy(v_hbm.at[pv_v.at[0]], ring.at[0], gsem)

    def pos_chunk(j, carry):
        par = lax.rem(j, 2)
        pltpu.make_async_copy(v_hbm.at[pl.ds(0, CH)], ring.at[par],
                              gsem).wait()

        @pl.when(j < POS_CH - 1)
        def _():
            pltpu.async_copy(v_hbm.at[pv_v.at[j + 1]],
                             ring.at[lax.rem(j + 1, 2)], gsem)

        def group(g, carry2):
            t0 = g * 16
            for s in range(16):
                t = t0 + s
                b = j * CH + t
                dot_store(pos_sb, b, b, ring, par, t)
            return carry2

        lax.fori_loop(0, CH // 16, group, 0)
        return carry

    lax.fori_loop(0, POS_CH, pos_chunk, 0)
    del pv0

    ng0 = pltpu.async_copy(v_hbm.at[ng_v.at[0]], ring.at[0], gsem)

    def neg_chunk(j, carry):
        par = lax.rem(j, 2)
        pltpu.make_async_copy(v_hbm.at[pl.ds(0, CH)], ring.at[par],
                              gsem).wait()

        @pl.when(j < NEG_CH - 1)
        def _():
            pltpu.async_copy(v_hbm.at[ng_v.at[j + 1]],
                             ring.at[lax.rem(j + 1, 2)], gsem)

        def group(g, carry2):
            r0 = j * CH + g * 16
            for s in range(16):
                r = r0 + s
                b = lax.div(r, N)
                dot_store(neg_sb, r, b, ring, par, r - j * CH)
            return carry2

        lax.fori_loop(0, CH // 16, group, 0)
        return carry

    lax.fori_loop(0, NEG_CH, neg_chunk, 0)
    del ng0

    pltpu.sync_copy(pos_sb, ps_out.at[wid])
    pltpu.sync_copy(neg_sb, ns_out.at[wid])
    wu.wait()


_sc_fused = functools.partial(
    pl.kernel,
    compiler_params=pltpu.CompilerParams(needs_layout_passes=False),
    out_type=[
        jax.ShapeDtypeStruct((B, W2), jnp.float32),
        jax.ShapeDtypeStruct((NW, BPW), jnp.float32),
        jax.ShapeDtypeStruct((NW, BPW * N), jnp.float32),
    ],
    mesh=plsc.VectorSubcoreMesh(core_axis_name="c", subcore_axis_name="s"),
    scratch_types=[
        pltpu.VMEM((BPW, W2), jnp.float32),
        pltpu.VMEM((2, CH, W2), jnp.float32),
        pltpu.VMEM((POS_CH, CH), jnp.int32),
        pltpu.VMEM((POS_CH, CH), jnp.int32),
        pltpu.VMEM((NEG_CH, CH), jnp.int32),
        pltpu.VMEM((BPW,), jnp.float32),
        pltpu.VMEM((BPW * N,), jnp.float32),
        pltpu.SemaphoreType.DMA,
        pltpu.SemaphoreType.DMA,
    ],
)(_sc_body)


_TR_BLK = 1024


def _tr_body(ut_ref, vt_ref, pu_ref, pv_ref, uv_ref):
    uv_ref[...] = (
        lax.dot_general(ut_ref[...], pu_ref[...], (((0,), (0,)), ((), ())),
                        preferred_element_type=jnp.float32)
        + lax.dot_general(vt_ref[...], pv_ref[...], (((0,), (0,)), ((), ())),
                          preferred_element_type=jnp.float32))


def _build_uv(u_emb, v_emb):
    ut = u_emb.T
    vt = v_emb.T
    r = jnp.arange(D)[:, None]
    c = jnp.arange(W2)[None, :]
    p_u = (c == r).astype(jnp.float32)
    p_v = (c == r + D).astype(jnp.float32)
    grid = (pl.cdiv(V, _TR_BLK),)
    return pl.pallas_call(
        _tr_body,
        grid=grid,
        in_specs=[
            pl.BlockSpec((D, _TR_BLK), lambda i: (0, i)),
            pl.BlockSpec((D, _TR_BLK), lambda i: (0, i)),
            pl.BlockSpec((D, W2), lambda i: (0, 0)),
            pl.BlockSpec((D, W2), lambda i: (0, 0)),
        ],
        out_specs=pl.BlockSpec((_TR_BLK, W2), lambda i: (i, 0)),
        out_shape=jax.ShapeDtypeStruct((V, W2), jnp.float32),
    )(ut, vt, p_u, p_v)


def _tc_body(eu_ref, ps_ref, ns_ref, w_ref, b_ref, acc_ref, dur_ref):
    s = jnp.clip(ps_ref[...], -10.0, 10.0)
    pos = jnp.log1p(jnp.exp(-s))
    ns = jnp.clip(ns_ref[...], -10.0, 10.0)
    neg = jnp.log1p(jnp.exp(ns))
    total = (jnp.sum(pos) + jnp.sum(neg)) * (1.0 / B)
    acc_ref[...] = total[None, None]
    dur_ref[...] = lax.dot_general(
        eu_ref[:, pl.ds(0, D)], w_ref[...], (((1,), (1,)), ((), ())),
        preferred_element_type=jnp.float32) + b_ref[...]


def _tc_math(eu_wide, pos_s, neg_s, W, b2):
    return pl.pallas_call(
        _tc_body,
        grid=(1,),
        in_specs=[
            pl.BlockSpec((B, W2), lambda i: (0, 0)),
            pl.BlockSpec((NW, BPW), lambda i: (0, 0)),
            pl.BlockSpec((NW, BPW * N), lambda i: (0, 0)),
            pl.BlockSpec((C, D), lambda i: (0, 0)),
            pl.BlockSpec((1, C), lambda i: (0, 0)),
        ],
        out_specs=[
            pl.BlockSpec((1, 1), lambda i: (0, 0)),
            pl.BlockSpec((B, C), lambda i: (0, 0)),
        ],
        out_shape=[
            jax.ShapeDtypeStruct((1, 1), jnp.float32),
            jax.ShapeDtypeStruct((B, C), jnp.float32),
        ],
    )(eu_wide, pos_s, neg_s, W, b2)


def kernel(pos_u, pos_v, neg_v, predict_fix, u_emb, v_emb, W, b):
    uv = _build_uv(u_emb, v_emb)
    pu = pos_u.astype(jnp.int32).reshape(NW, POS_CH, CH)
    pv = pos_v.astype(jnp.int32).reshape(NW, POS_CH, CH)
    ng = neg_v.astype(jnp.int32).reshape(NW, NEG_CH, CH)

    eu_wide, pos_s, neg_s = _sc_fused(pu, pv, ng, uv, uv)

    acc, duration = _tc_math(eu_wide, pos_s, neg_s, W, b.reshape(1, C))
    return acc[0, 0], duration


# speedup vs baseline: 1.7811x; 1.6035x over previous
import functools

import jax
import jax.numpy as jnp
from jax import lax
from jax.experimental import pallas as pl
from jax.experimental.pallas import tpu as pltpu
from jax.experimental.pallas import tpu_sc as plsc

V, D, C, B, N = 1000000, 64, 16, 16384, 20

NC, NS = 2, 16
NW = NC * NS
CH = 128
BPW = B // NW
POS_CH = BPW // CH
NEG_CH = BPW * N // CH
W2 = 2 * D


def _sc_body(pu_hbm, pv_hbm, ng_hbm, u_hbm, v_hbm,
             eu_out, ps_out, ns_out,
             u_wide, ring, pu_v, pv_v, ng_v, pos_sb, neg_sb, gsem, wsem):
    wid = lax.axis_index("s") * NC + lax.axis_index("c")
    pltpu.sync_copy(pu_hbm.at[wid], pu_v)
    pltpu.sync_copy(pv_hbm.at[wid], pv_v)
    pltpu.sync_copy(ng_hbm.at[wid], ng_v)

    lane = lax.iota(jnp.int32, 16)
    last = lane == 15

    hu = [pltpu.async_copy(u_hbm.at[pu_v.at[j]],
                           u_wide.at[pl.ds(j * CH, CH)], gsem)
          for j in range(POS_CH)]
    for c in hu:
        c.wait()
    wu = pltpu.async_copy(u_wide, eu_out.at[pl.ds(wid * BPW, BPW)], wsem)

    def dot_store(sb, store_idx, b_row, buf, parity, t):
        acc = u_wide[b_row, pl.ds(0, 16)] * buf[parity, t, pl.ds(D, 16)]
        for k in range(1, 4):
            acc += (u_wide[b_row, pl.ds(k * 16, 16)]
                    * buf[parity, t, pl.ds(D + k * 16, 16)])
        tot = plsc.cumsum(acc)
        plsc.store_scatter(sb, [jnp.full((16,), store_idx, jnp.int32)], tot,
                           mask=last)

    pv0 = pltpu.async_copy(v_hbm.at[pv_v.at[0]], ring.at[0], gsem)

    def pos_chunk(j, carry):
        par = lax.rem(j, 2)
        pltpu.make_async_copy(v_hbm.at[pl.ds(0, CH)], ring.at[par],
                              gsem).wait()

        @pl.when(j < POS_CH - 1)
        def _():
            pltpu.async_copy(v_hbm.at[pv_v.at[j + 1]],
                             ring.at[lax.rem(j + 1, 2)], gsem)

        def group(g, carry2):
            t0 = g * 16
            for s in range(16):
                t = t0 + s
                b = j * CH + t
                dot_store(pos_sb, b, b, ring, par, t)
            return carry2

        lax.fori_loop(0, CH // 16, group, 0)
        return carry

    lax.fori_loop(0, POS_CH, pos_chunk, 0)
    del pv0

    ng0 = pltpu.async_copy(v_hbm.at[ng_v.at[0]], ring.at[0], gsem)

    def neg_chunk(j, carry):
        par = lax.rem(j, 2)
        pltpu.make_async_copy(v_hbm.at[pl.ds(0, CH)], ring.at[par],
                              gsem).wait()

        @pl.when(j < NEG_CH - 1)
        def _():
            pltpu.async_copy(v_hbm.at[ng_v.at[j + 1]],
                             ring.at[lax.rem(j + 1, 2)], gsem)

        def group(g, carry2):
            r0 = j * CH + g * 16
            for s in range(16):
                r = r0 + s
                b = lax.div(r, N)
                dot_store(neg_sb, r, b, ring, par, r - j * CH)
            return carry2

        lax.fori_loop(0, CH // 16, group, 0)
        return carry

    lax.fori_loop(0, NEG_CH, neg_chunk, 0)
    del ng0

    pltpu.sync_copy(pos_sb, ps_out.at[wid])
    pltpu.sync_copy(neg_sb, ns_out.at[wid])
    wu.wait()


_sc_fused = functools.partial(
    pl.kernel,
    compiler_params=pltpu.CompilerParams(needs_layout_passes=False),
    out_type=[
        jax.ShapeDtypeStruct((B, W2), jnp.float32),
        jax.ShapeDtypeStruct((NW, BPW), jnp.float32),
        jax.ShapeDtypeStruct((NW, BPW * N), jnp.float32),
    ],
    mesh=plsc.VectorSubcoreMesh(core_axis_name="c", subcore_axis_name="s"),
    scratch_types=[
        pltpu.VMEM((BPW, W2), jnp.float32),
        pltpu.VMEM((2, CH, W2), jnp.float32),
        pltpu.VMEM((POS_CH, CH), jnp.int32),
        pltpu.VMEM((POS_CH, CH), jnp.int32),
        pltpu.VMEM((NEG_CH, CH), jnp.int32),
        pltpu.VMEM((BPW,), jnp.float32),
        pltpu.VMEM((BPW * N,), jnp.float32),
        pltpu.SemaphoreType.DMA,
        pltpu.SemaphoreType.DMA,
    ],
)(_sc_body)


_TR_BLK = 4096


def _tr_body(ut_ref, vt_ref, pu_ref, pv_ref, uv_ref):
    uv_ref[...] = (
        lax.dot_general(ut_ref[...], pu_ref[...], (((0,), (0,)), ((), ())),
                        preferred_element_type=jnp.float32)
        + lax.dot_general(vt_ref[...], pv_ref[...], (((0,), (0,)), ((), ())),
                          preferred_element_type=jnp.float32))


def _build_uv(u_emb, v_emb):
    ut = u_emb.T
    vt = v_emb.T
    r = jnp.arange(D)[:, None]
    c = jnp.arange(W2)[None, :]
    p_u = (c == r).astype(jnp.float32)
    p_v = (c == r + D).astype(jnp.float32)
    grid = (pl.cdiv(V, _TR_BLK),)
    return pl.pallas_call(
        _tr_body,
        grid=grid,
        in_specs=[
            pl.BlockSpec((D, _TR_BLK), lambda i: (0, i)),
            pl.BlockSpec((D, _TR_BLK), lambda i: (0, i)),
            pl.BlockSpec((D, W2), lambda i: (0, 0)),
            pl.BlockSpec((D, W2), lambda i: (0, 0)),
        ],
        out_specs=pl.BlockSpec((_TR_BLK, W2), lambda i: (i, 0)),
        out_shape=jax.ShapeDtypeStruct((V, W2), jnp.float32),
    )(ut, vt, p_u, p_v)


def _tc_body(eu_ref, ps_ref, ns_ref, w_ref, b_ref, acc_ref, dur_ref):
    s = jnp.clip(ps_ref[...], -10.0, 10.0)
    pos = jnp.log1p(jnp.exp(-s))
    ns = jnp.clip(ns_ref[...], -10.0, 10.0)
    neg = jnp.log1p(jnp.exp(ns))
    total = (jnp.sum(pos) + jnp.sum(neg)) * (1.0 / B)
    acc_ref[...] = total[None, None]
    dur_ref[...] = lax.dot_general(
        eu_ref[:, pl.ds(0, D)], w_ref[...], (((1,), (1,)), ((), ())),
        preferred_element_type=jnp.float32) + b_ref[...]


def _tc_math(eu_wide, pos_s, neg_s, W, b2):
    return pl.pallas_call(
        _tc_body,
        grid=(1,),
        in_specs=[
            pl.BlockSpec((B, W2), lambda i: (0, 0)),
            pl.BlockSpec((NW, BPW), lambda i: (0, 0)),
            pl.BlockSpec((NW, BPW * N), lambda i: (0, 0)),
            pl.BlockSpec((C, D), lambda i: (0, 0)),
            pl.BlockSpec((1, C), lambda i: (0, 0)),
        ],
        out_specs=[
            pl.BlockSpec((1, 1), lambda i: (0, 0)),
            pl.BlockSpec((B, C), lambda i: (0, 0)),
        ],
        out_shape=[
            jax.ShapeDtypeStruct((1, 1), jnp.float32),
            jax.ShapeDtypeStruct((B, C), jnp.float32),
        ],
    )(eu_wide, pos_s, neg_s, W, b2)


def kernel(pos_u, pos_v, neg_v, predict_fix, u_emb, v_emb, W, b):
    uv = _build_uv(u_emb, v_emb)
    pu = pos_u.astype(jnp.int32).reshape(NW, POS_CH, CH)
    pv = pos_v.astype(jnp.int32).reshape(NW, POS_CH, CH)
    ng = neg_v.astype(jnp.int32).reshape(NW, NEG_CH, CH)

    eu_wide, pos_s, neg_s = _sc_fused(pu, pv, ng, uv, uv)

    acc, duration = _tc_math(eu_wide, pos_s, neg_s, W, b.reshape(1, C))
    return acc[0, 0], duration


# speedup vs baseline: 2.0024x; 1.1242x over previous
import functools

import jax
import jax.numpy as jnp
from jax import lax
from jax.experimental import pallas as pl
from jax.experimental.pallas import tpu as pltpu
from jax.experimental.pallas import tpu_sc as plsc

V, D, C, B, N = 1000000, 64, 16, 16384, 20

NC, NS = 2, 16
NW = NC * NS
CH = 128
BPW = B // NW
POS_CH = BPW // CH
NEG_CH = BPW * N // CH
W2 = 2 * D


def _sc_body(pu_hbm, pv_hbm, ng_hbm, u_hbm, v_hbm,
             eu_out, ps_out, ns_out,
             u_wide, ring, pu_v, pv_v, ng_v, pos_sb, neg_sb, gsem, wsem):
    wid = lax.axis_index("s") * NC + lax.axis_index("c")
    pltpu.sync_copy(pu_hbm.at[wid], pu_v)
    pltpu.sync_copy(pv_hbm.at[wid], pv_v)
    pltpu.sync_copy(ng_hbm.at[wid], ng_v)

    lane = lax.iota(jnp.int32, 16)
    last = lane == 15

    hu = [pltpu.async_copy(u_hbm.at[pu_v.at[j]],
                           u_wide.at[pl.ds(j * CH, CH)], gsem)
          for j in range(POS_CH)]
    for c in hu:
        c.wait()
    wu = pltpu.async_copy(u_wide, eu_out.at[pl.ds(wid * BPW, BPW)], wsem)

    def dot_store(sb, store_idx, b_row, buf, parity, t):
        acc = u_wide[b_row, pl.ds(0, 16)] * buf[parity, t, pl.ds(D, 16)]
        for k in range(1, 4):
            acc += (u_wide[b_row, pl.ds(k * 16, 16)]
                    * buf[parity, t, pl.ds(D + k * 16, 16)])
        tot = plsc.cumsum(acc)
        plsc.store_scatter(sb, [jnp.full((16,), store_idx, jnp.int32)], tot,
                           mask=last)

    pv0 = pltpu.async_copy(v_hbm.at[pv_v.at[0]], ring.at[0], gsem)

    def pos_chunk(j, carry):
        par = lax.rem(j, 2)
        pltpu.make_async_copy(v_hbm.at[pl.ds(0, CH)], ring.at[par],
                              gsem).wait()

        @pl.when(j < POS_CH - 1)
        def _():
            pltpu.async_copy(v_hbm.at[pv_v.at[j + 1]],
                             ring.at[lax.rem(j + 1, 2)], gsem)

        def group(g, carry2):
            t0 = g * 16
            for s in range(16):
                t = t0 + s
                b = j * CH + t
                dot_store(pos_sb, b, b, ring, par, t)
            return carry2

        lax.fori_loop(0, CH // 16, group, 0)
        return carry

    lax.fori_loop(0, POS_CH, pos_chunk, 0)
    del pv0

    ng0 = pltpu.async_copy(v_hbm.at[ng_v.at[0]], ring.at[0], gsem)

    def neg_chunk(j, carry):
        par = lax.rem(j, 2)
        pltpu.make_async_copy(v_hbm.at[pl.ds(0, CH)], ring.at[par],
                              gsem).wait()

        @pl.when(j < NEG_CH - 1)
        def _():
            pltpu.async_copy(v_hbm.at[ng_v.at[j + 1]],
                             ring.at[lax.rem(j + 1, 2)], gsem)

        def group(g, carry2):
            r0 = j * CH + g * 16
            for s in range(16):
                r = r0 + s
                b = lax.div(r, N)
                dot_store(neg_sb, r, b, ring, par, r - j * CH)
            return carry2

        lax.fori_loop(0, CH // 16, group, 0)
        return carry

    lax.fori_loop(0, NEG_CH, neg_chunk, 0)
    del ng0

    pltpu.sync_copy(pos_sb, ps_out.at[wid])
    pltpu.sync_copy(neg_sb, ns_out.at[wid])
    wu.wait()


_sc_fused = functools.partial(
    pl.kernel,
    compiler_params=pltpu.CompilerParams(needs_layout_passes=False),
    out_type=[
        jax.ShapeDtypeStruct((B, W2), jnp.float32),
        jax.ShapeDtypeStruct((NW, BPW), jnp.float32),
        jax.ShapeDtypeStruct((NW, BPW * N), jnp.float32),
    ],
    mesh=plsc.VectorSubcoreMesh(core_axis_name="c", subcore_axis_name="s"),
    scratch_types=[
        pltpu.VMEM((BPW, W2), jnp.float32),
        pltpu.VMEM((2, CH, W2), jnp.float32),
        pltpu.VMEM((POS_CH, CH), jnp.int32),
        pltpu.VMEM((POS_CH, CH), jnp.int32),
        pltpu.VMEM((NEG_CH, CH), jnp.int32),
        pltpu.VMEM((BPW,), jnp.float32),
        pltpu.VMEM((BPW * N,), jnp.float32),
        pltpu.SemaphoreType.DMA,
        pltpu.SemaphoreType.DMA,
    ],
)(_sc_body)


_TR_BLK = 8192


def _tr_body(ut_ref, vt_ref, pu_ref, pv_ref, uv_ref):
    uv_ref[...] = (
        lax.dot_general(ut_ref[...], pu_ref[...], (((0,), (0,)), ((), ())),
                        preferred_element_type=jnp.float32)
        + lax.dot_general(vt_ref[...], pv_ref[...], (((0,), (0,)), ((), ())),
                          preferred_element_type=jnp.float32))


def _build_uv(u_emb, v_emb):
    ut = u_emb.T
    vt = v_emb.T
    r = jnp.arange(D)[:, None]
    c = jnp.arange(W2)[None, :]
    p_u = (c == r).astype(jnp.float32)
    p_v = (c == r + D).astype(jnp.float32)
    grid = (pl.cdiv(V, _TR_BLK),)
    return pl.pallas_call(
        _tr_body,
        grid=grid,
        in_specs=[
            pl.BlockSpec((D, _TR_BLK), lambda i: (0, i)),
            pl.BlockSpec((D, _TR_BLK), lambda i: (0, i)),
            pl.BlockSpec((D, W2), lambda i: (0, 0)),
            pl.BlockSpec((D, W2), lambda i: (0, 0)),
        ],
        out_specs=pl.BlockSpec((_TR_BLK, W2), lambda i: (i, 0)),
        out_shape=jax.ShapeDtypeStruct((V, W2), jnp.float32),
    )(ut, vt, p_u, p_v)


def _tc_body(eu_ref, ps_ref, ns_ref, w_ref, b_ref, acc_ref, dur_ref):
    s = jnp.clip(ps_ref[...], -10.0, 10.0)
    pos = jnp.log1p(jnp.exp(-s))
    ns = jnp.clip(ns_ref[...], -10.0, 10.0)
    neg = jnp.log1p(jnp.exp(ns))
    total = (jnp.sum(pos) + jnp.sum(neg)) * (1.0 / B)
    acc_ref[...] = total[None, None]
    dur_ref[...] = lax.dot_general(
        eu_ref[:, pl.ds(0, D)], w_ref[...], (((1,), (1,)), ((), ())),
        preferred_element_type=jnp.float32) + b_ref[...]


def _tc_math(eu_wide, pos_s, neg_s, W, b2):
    return pl.pallas_call(
        _tc_body,
        grid=(1,),
        in_specs=[
            pl.BlockSpec((B, W2), lambda i: (0, 0)),
            pl.BlockSpec((NW, BPW), lambda i: (0, 0)),
            pl.BlockSpec((NW, BPW * N), lambda i: (0, 0)),
            pl.BlockSpec((C, D), lambda i: (0, 0)),
            pl.BlockSpec((1, C), lambda i: (0, 0)),
        ],
        out_specs=[
            pl.BlockSpec((1, 1), lambda i: (0, 0)),
            pl.BlockSpec((B, C), lambda i: (0, 0)),
        ],
        out_shape=[
            jax.ShapeDtypeStruct((1, 1), jnp.float32),
            jax.ShapeDtypeStruct((B, C), jnp.float32),
        ],
    )(eu_wide, pos_s, neg_s, W, b2)


def kernel(pos_u, pos_v, neg_v, predict_fix, u_emb, v_emb, W, b):
    uv = _build_uv(u_emb, v_emb)
    pu = pos_u.astype(jnp.int32).reshape(NW, POS_CH, CH)
    pv = pos_v.astype(jnp.int32).reshape(NW, POS_CH, CH)
    ng = neg_v.astype(jnp.int32).reshape(NW, NEG_CH, CH)

    eu_wide, pos_s, neg_s = _sc_fused(pu, pv, ng, uv, uv)

    acc, duration = _tc_math(eu_wide, pos_s, neg_s, W, b.reshape(1, C))
    return acc[0, 0], duration


# speedup vs baseline: 2.1418x; 1.0696x over previous
import functools

import jax
import jax.numpy as jnp
from jax import lax
from jax.experimental import pallas as pl
from jax.experimental.pallas import tpu as pltpu
from jax.experimental.pallas import tpu_sc as plsc

V, D, C, B, N = 1000000, 64, 16, 16384, 20

NC, NS = 2, 16
NW = NC * NS
CH = 128
BPW = B // NW
POS_CH = BPW // CH
NEG_CH = BPW * N // CH
W2 = 2 * D


def _sc_body(pu_hbm, pv_hbm, ng_hbm, u_hbm, v_hbm,
             eu_out, ps_out, ns_out,
             u_wide, ring, pu_v, pv_v, ng_v, pos_sb, neg_sb, gsem, wsem):
    wid = lax.axis_index("s") * NC + lax.axis_index("c")
    pltpu.sync_copy(pu_hbm.at[wid], pu_v)
    pltpu.sync_copy(pv_hbm.at[wid], pv_v)
    pltpu.sync_copy(ng_hbm.at[wid], ng_v)

    lane = lax.iota(jnp.int32, 16)
    last = lane == 15

    hu = [pltpu.async_copy(u_hbm.at[pu_v.at[j]],
                           u_wide.at[pl.ds(j * CH, CH)], gsem)
          for j in range(POS_CH)]
    for c in hu:
        c.wait()
    wu = pltpu.async_copy(u_wide, eu_out.at[pl.ds(wid * BPW, BPW)], wsem)

    def dot_store(sb, store_idx, b_row, buf, parity, t):
        acc = u_wide[b_row, pl.ds(0, 16)] * buf[parity, t, pl.ds(D, 16)]
        for k in range(1, 4):
            acc += (u_wide[b_row, pl.ds(k * 16, 16)]
                    * buf[parity, t, pl.ds(D + k * 16, 16)])
        tot = plsc.cumsum(acc)
        plsc.store_scatter(sb, [jnp.full((16,), store_idx, jnp.int32)], tot,
                           mask=last)

    pv0 = pltpu.async_copy(v_hbm.at[pv_v.at[0]], ring.at[0], gsem)

    def pos_chunk(j, carry):
        par = lax.rem(j, 2)
        pltpu.make_async_copy(v_hbm.at[pl.ds(0, CH)], ring.at[par],
                              gsem).wait()

        @pl.when(j < POS_CH - 1)
        def _():
            pltpu.async_copy(v_hbm.at[pv_v.at[j + 1]],
                             ring.at[lax.rem(j + 1, 2)], gsem)

        def group(g, carry2):
            t0 = g * 16
            for s in range(16):
                t = t0 + s
                b = j * CH + t
                dot_store(pos_sb, b, b, ring, par, t)
            return carry2

        lax.fori_loop(0, CH // 16, group, 0)
        return carry

    lax.fori_loop(0, POS_CH, pos_chunk, 0)
    del pv0

    ng0 = pltpu.async_copy(v_hbm.at[ng_v.at[0]], ring.at[0], gsem)

    def neg_chunk(j, carry):
        par = lax.rem(j, 2)
        pltpu.make_async_copy(v_hbm.at[pl.ds(0, CH)], ring.at[par],
                              gsem).wait()

        @pl.when(j < NEG_CH - 1)
        def _():
            pltpu.async_copy(v_hbm.at[ng_v.at[j + 1]],
                             ring.at[lax.rem(j + 1, 2)], gsem)

        def group(g, carry2):
            r0 = j * CH + g * 16
            for s in range(16):
                r = r0 + s
                b = lax.div(r, N)
                dot_store(neg_sb, r, b, ring, par, r - j * CH)
            return carry2

        lax.fori_loop(0, CH // 16, group, 0)
        return carry

    lax.fori_loop(0, NEG_CH, neg_chunk, 0)
    del ng0

    pltpu.sync_copy(pos_sb, ps_out.at[wid])
    pltpu.sync_copy(neg_sb, ns_out.at[wid])
    wu.wait()


_sc_fused = functools.partial(
    pl.kernel,
    compiler_params=pltpu.CompilerParams(needs_layout_passes=False),
    out_type=[
        jax.ShapeDtypeStruct((B, W2), jnp.float32),
        jax.ShapeDtypeStruct((NW, BPW), jnp.float32),
        jax.ShapeDtypeStruct((NW, BPW * N), jnp.float32),
    ],
    mesh=plsc.VectorSubcoreMesh(core_axis_name="c", subcore_axis_name="s"),
    scratch_types=[
        pltpu.VMEM((BPW, W2), jnp.float32),
        pltpu.VMEM((2, CH, W2), jnp.float32),
        pltpu.VMEM((POS_CH, CH), jnp.int32),
        pltpu.VMEM((POS_CH, CH), jnp.int32),
        pltpu.VMEM((NEG_CH, CH), jnp.int32),
        pltpu.VMEM((BPW,), jnp.float32),
        pltpu.VMEM((BPW * N,), jnp.float32),
        pltpu.SemaphoreType.DMA,
        pltpu.SemaphoreType.DMA,
    ],
)(_sc_body)


_TR_BLK = 16384


def _tr_body(ut_ref, vt_ref, pu_ref, pv_ref, uv_ref):
    uv_ref[...] = (
        lax.dot_general(ut_ref[...], pu_ref[...], (((0,), (0,)), ((), ())),
                        preferred_element_type=jnp.float32)
        + lax.dot_general(vt_ref[...], pv_ref[...], (((0,), (0,)), ((), ())),
                          preferred_element_type=jnp.float32))


def _build_uv(u_emb, v_emb):
    ut = u_emb.T
    vt = v_emb.T
    r = jnp.arange(D)[:, None]
    c = jnp.arange(W2)[None, :]
    p_u = (c == r).astype(jnp.float32)
    p_v = (c == r + D).astype(jnp.float32)
    grid = (pl.cdiv(V, _TR_BLK),)
    return pl.pallas_call(
        _tr_body,
        grid=grid,
        in_specs=[
            pl.BlockSpec((D, _TR_BLK), lambda i: (0, i)),
            pl.BlockSpec((D, _TR_BLK), lambda i: (0, i)),
            pl.BlockSpec((D, W2), lambda i: (0, 0)),
            pl.BlockSpec((D, W2), lambda i: (0, 0)),
        ],
        out_specs=pl.BlockSpec((_TR_BLK, W2), lambda i: (i, 0)),
        out_shape=jax.ShapeDtypeStruct((V, W2), jnp.float32),
    )(ut, vt, p_u, p_v)


def _tc_body(eu_ref, ps_ref, ns_ref, w_ref, b_ref, acc_ref, dur_ref):
    s = jnp.clip(ps_ref[...], -10.0, 10.0)
    pos = jnp.log1p(jnp.exp(-s))
    ns = jnp.clip(ns_ref[...], -10.0, 10.0)
    neg = jnp.log1p(jnp.exp(ns))
    total = (jnp.sum(pos) + jnp.sum(neg)) * (1.0 / B)
    acc_ref[...] = total[None, None]
    dur_ref[...] = lax.dot_general(
        eu_ref[:, pl.ds(0, D)], w_ref[...], (((1,), (1,)), ((), ())),
        preferred_element_type=jnp.float32) + b_ref[...]


def _tc_math(eu_wide, pos_s, neg_s, W, b2):
    return pl.pallas_call(
        _tc_body,
        grid=(1,),
        in_specs=[
            pl.BlockSpec((B, W2), lambda i: (0, 0)),
            pl.BlockSpec((NW, BPW), lambda i: (0, 0)),
            pl.BlockSpec((NW, BPW * N), lambda i: (0, 0)),
            pl.BlockSpec((C, D), lambda i: (0, 0)),
            pl.BlockSpec((1, C), lambda i: (0, 0)),
        ],
        out_specs=[
            pl.BlockSpec((1, 1), lambda i: (0, 0)),
            pl.BlockSpec((B, C), lambda i: (0, 0)),
        ],
        out_shape=[
            jax.ShapeDtypeStruct((1, 1), jnp.float32),
            jax.ShapeDtypeStruct((B, C), jnp.float32),
        ],
    )(eu_wide, pos_s, neg_s, W, b2)


def kernel(pos_u, pos_v, neg_v, predict_fix, u_emb, v_emb, W, b):
    uv = _build_uv(u_emb, v_emb)
    pu = pos_u.astype(jnp.int32).reshape(NW, POS_CH, CH)
    pv = pos_v.astype(jnp.int32).reshape(NW, POS_CH, CH)
    ng = neg_v.astype(jnp.int32).reshape(NW, NEG_CH, CH)

    eu_wide, pos_s, neg_s = _sc_fused(pu, pv, ng, uv, uv)

    acc, duration = _tc_math(eu_wide, pos_s, neg_s, W, b.reshape(1, C))
    return acc[0, 0], duration
